# Initial kernel scaffold; baseline (speedup 1.0000x reference)
import functools

import jax
import jax.numpy as jnp
from jax import lax
from jax.experimental import pallas as pl
from jax.experimental.pallas import tpu as pltpu
from jax.experimental.pallas import tpu_sc as plsc

_NC, _NS, _L = 2, 16, 16
_NW = _NC * _NS
_P = 128


@functools.partial(jax.jit, static_argnums=(3, 4, 5, 6, 7))
def _sc_interp(table, posx, posy, B, C, H, W, N):
    NP = B * N
    n_chunks = NP // _P
    assert NP % _P == 0 and C % _L == 0
    HW = H * W
    fH, fW = float(H), float(W)
    sx = float(max(W - 1, 1))
    sy = float(max(H - 1, 1))
    mesh = plsc.VectorSubcoreMesh(core_axis_name="c", subcore_axis_name="s")

    @functools.partial(
        pl.kernel,
        out_type=jax.ShapeDtypeStruct((NP, C), jnp.float32),
        mesh=mesh,
        scratch_types=dict(
            px_v=pltpu.VMEM((_P,), jnp.float32),
            py_v=pltpu.VMEM((_P,), jnp.float32),
            idx_v=[pltpu.VMEM((_P,), jnp.int32) for _ in range(4)],
            w_v=[pltpu.VMEM((_P,), jnp.float32) for _ in range(4)],
            g_v=[pltpu.VMEM((_P, C), jnp.float32) for _ in range(4)],
            o_v=pltpu.VMEM((_P, C), jnp.float32),
            sems=[pltpu.SemaphoreType.DMA for _ in range(4)],
        ),
    )
    def kern(table_hbm, posx_hbm, posy_hbm, out_hbm,
             px_v, py_v, idx_v, w_v, g_v, o_v, sems):
        wid = lax.axis_index("s") * _NC + lax.axis_index("c")
        lane = lax.iota(jnp.int32, 16)

        def chunk_body(i, _):
            c = wid + i * _NW
            base = c * _P
            pltpu.sync_copy(posx_hbm.at[pl.ds(base, _P)], px_v)
            pltpu.sync_copy(posy_hbm.at[pl.ds(base, _P)], py_v)

            for j in range(_P // _L):
                s = pl.ds(j * _L, _L)
                px = px_v[s]
                py = py_v[s]
                gx = 2.0 * (px / sx) - 1.0
                gy = 2.0 * (py / sy) - 1.0
                ix = ((gx + 1.0) * fW - 1.0) / 2.0
                iy = ((gy + 1.0) * fH - 1.0) / 2.0
                tx = ix.astype(jnp.int32)
                tx = jnp.where(tx.astype(jnp.float32) > ix, tx - 1, tx)
                ty = iy.astype(jnp.int32)
                ty = jnp.where(ty.astype(jnp.float32) > iy, ty - 1, ty)
                wx1 = ix - tx.astype(jnp.float32)
                wx0 = 1.0 - wx1
                wy1 = iy - ty.astype(jnp.float32)
                wy0 = 1.0 - wy1
                x1 = tx + 1
                y1 = ty + 1
                okx0 = (tx >= 0) & (tx < W)
                okx1 = (x1 >= 0) & (x1 < W)
                oky0 = (ty >= 0) & (ty < H)
                oky1 = (y1 >= 0) & (y1 < H)
                x0c = jnp.clip(tx, 0, W - 1)
                x1c = jnp.clip(x1, 0, W - 1)
                y0c = jnp.clip(ty, 0, H - 1)
                y1c = jnp.clip(y1, 0, H - 1)
                p_vec = base + j * _L + lane
                rowbase = jnp.zeros((16,), jnp.int32)
                for k in range(1, B):
                    rowbase = rowbase + jnp.where(p_vec >= k * N, HW, 0)
                r0 = rowbase + y0c * W
                r1 = rowbase + y1c * W
                idx_v[0][s] = r0 + x0c
                idx_v[1][s] = r0 + x1c
                idx_v[2][s] = r1 + x0c
                idx_v[3][s] = r1 + x1c
                w_v[0][s] = jnp.where(okx0 & oky0, wx0 * wy0, 0.0)
                w_v[1][s] = jnp.where(okx1 & oky0, wx1 * wy0, 0.0)
                w_v[2][s] = jnp.where(okx0 & oky1, wx0 * wy1, 0.0)
                w_v[3][s] = jnp.where(okx1 & oky1, wx1 * wy1, 0.0)

            cps = [pltpu.async_copy(table_hbm.at[idx_v[k]], g_v[k], sems[k])
                   for k in range(4)]
            for cp in cps:
                cp.wait()

            def pt_body(p, _):
                a00 = w_v[0][p]
                a10 = w_v[1][p]
                a01 = w_v[2][p]
                a11 = w_v[3][p]
                for q in range(C // _L):
                    sq = pl.ds(q * _L, _L)
                    o_v[p, sq] = (g_v[0][p, sq] * a00 + g_v[1][p, sq] * a10
                                  + g_v[2][p, sq] * a01 + g_v[3][p, sq] * a11)
                return 0

            lax.fori_loop(0, _P, pt_body, 0)
            pltpu.sync_copy(o_v, out_hbm.at[pl.ds(base, _P)])
            return 0

        n_w = lax.div(n_chunks - wid + _NW - 1, _NW)
        lax.fori_loop(0, n_w, chunk_body, 0)

    return kern(table, posx, posy)


def kernel(x, pos, height, width):
    B, C, H, W = x.shape
    N = pos.shape[1]
    table = jnp.transpose(x, (0, 2, 3, 1)).reshape(B * H * W, C)
    posx = pos[:, :, 0].reshape(-1)
    posy = pos[:, :, 1].reshape(-1)
    out = _sc_interp(table, posx, posy, B, C, H, W, N)
    return out.reshape(B, N, C)


# speedup vs baseline: 1.3905x; 1.3905x over previous
import functools

import jax
import jax.numpy as jnp
from jax import lax
from jax.experimental import pallas as pl
from jax.experimental.pallas import tpu as pltpu
from jax.experimental.pallas import tpu_sc as plsc

_NC, _NS, _L = 2, 16, 16
_NW = _NC * _NS
_P = 128


@functools.partial(jax.jit, static_argnums=(3, 4, 5, 6, 7))
def _sc_interp(pair_table, posx, posy, B, C, H, W, N):
    NP = B * N
    n_chunks = NP // _P
    assert NP % _P == 0 and C % _L == 0
    HW = H * W
    fH, fW = float(H), float(W)
    sx = float(max(W - 1, 1))
    sy = float(max(H - 1, 1))
    mesh = plsc.VectorSubcoreMesh(core_axis_name="c", subcore_axis_name="s")

    @functools.partial(
        pl.kernel,
        out_type=jax.ShapeDtypeStruct((NP, C), jnp.float32),
        mesh=mesh,
        scratch_types=dict(
            px_v=pltpu.VMEM((_P,), jnp.float32),
            py_v=pltpu.VMEM((_P,), jnp.float32),
            idx_v=[pltpu.VMEM((_P,), jnp.int32) for _ in range(2)],
            w_v=[pltpu.VMEM((_P,), jnp.float32) for _ in range(4)],
            g_v=[pltpu.VMEM((_P, 2 * C), jnp.float32) for _ in range(2)],
            o_v=pltpu.VMEM((_P, C), jnp.float32),
            sems=[pltpu.SemaphoreType.DMA for _ in range(2)],
        ),
    )
    def kern(table_hbm, posx_hbm, posy_hbm, out_hbm,
             px_v, py_v, idx_v, w_v, g_v, o_v, sems):
        wid = lax.axis_index("s") * _NC + lax.axis_index("c")
        lane = lax.iota(jnp.int32, 16)

        def chunk_body(i, _):
            c = wid + i * _NW
            base = c * _P
            pltpu.sync_copy(posx_hbm.at[pl.ds(base, _P)], px_v)
            pltpu.sync_copy(posy_hbm.at[pl.ds(base, _P)], py_v)

            for j in range(_P // _L):
                s = pl.ds(j * _L, _L)
                px = px_v[s]
                py = py_v[s]
                gx = 2.0 * (px / sx) - 1.0
                gy = 2.0 * (py / sy) - 1.0
                ix = ((gx + 1.0) * fW - 1.0) / 2.0
                iy = ((gy + 1.0) * fH - 1.0) / 2.0
                tx = ix.astype(jnp.int32)
                tx = jnp.where(tx.astype(jnp.float32) > ix, tx - 1, tx)
                ty = iy.astype(jnp.int32)
                ty = jnp.where(ty.astype(jnp.float32) > iy, ty - 1, ty)
                wx1 = ix - tx.astype(jnp.float32)
                wx0 = 1.0 - wx1
                wy1 = iy - ty.astype(jnp.float32)
                wy0 = 1.0 - wy1
                x1 = tx + 1
                y1 = ty + 1
                ax = (jnp.where((tx >= 0) & (tx < W), wx0, 0.0)
                      + jnp.where(tx == -1, wx1, 0.0))
                bx = jnp.where((tx >= 0) & (x1 < W), wx1, 0.0)
                ay0 = jnp.where((ty >= 0) & (ty < H), wy0, 0.0)
                ay1 = jnp.where((y1 >= 0) & (y1 < H), wy1, 0.0)
                x0c = jnp.clip(tx, 0, W - 1)
                y0c = jnp.clip(ty, 0, H - 1)
                y1c = jnp.clip(y1, 0, H - 1)
                p_vec = base + j * _L + lane
                rowbase = jnp.zeros((16,), jnp.int32)
                for k in range(1, B):
                    rowbase = rowbase + jnp.where(p_vec >= k * N, HW, 0)
                idx_v[0][s] = rowbase + y0c * W + x0c
                idx_v[1][s] = rowbase + y1c * W + x0c
                w_v[0][s] = ax * ay0
                w_v[1][s] = bx * ay0
                w_v[2][s] = ax * ay1
                w_v[3][s] = bx * ay1

            cps = [pltpu.async_copy(table_hbm.at[idx_v[k]], g_v[k], sems[k])
                   for k in range(2)]
            for cp in cps:
                cp.wait()

            def grp_body(jv, _):
                gbase = jv * _L
                sg = pl.ds(gbase, _L)
                wv = [w_v[k][sg] for k in range(4)]
                for ii in range(_L):
                    p = gbase + ii
                    a00, a10, a01, a11 = wv[0][ii], wv[1][ii], wv[2][ii], wv[3][ii]
                    for q in range(C // _L):
                        s0 = pl.ds(q * _L, _L)
                        s1 = pl.ds(C + q * _L, _L)
                        o_v[p, s0] = (g_v[0][p, s0] * a00 + g_v[0][p, s1] * a10
                                      + g_v[1][p, s0] * a01 + g_v[1][p, s1] * a11)
                return 0

            lax.fori_loop(0, _P // _L, grp_body, 0)
            pltpu.sync_copy(o_v, out_hbm.at[pl.ds(base, _P)])
            return 0

        n_w = lax.div(n_chunks - wid + _NW - 1, _NW)
        lax.fori_loop(0, n_w, chunk_body, 0)

    return kern(pair_table, posx, posy)


def kernel(x, pos, height, width):
    B, C, H, W = x.shape
    N = pos.shape[1]
    t = jnp.transpose(x, (0, 2, 3, 1)).reshape(B * H * W, C)
    pair_table = jnp.concatenate([t, jnp.roll(t, -1, axis=0)], axis=1)
    posx = pos[:, :, 0].reshape(-1)
    posy = pos[:, :, 1].reshape(-1)
    out = _sc_interp(pair_table, posx, posy, B, C, H, W, N)
    return out.reshape(B, N, C)


# speedup vs baseline: 2.0056x; 1.4424x over previous
import functools

import jax
import jax.numpy as jnp
from jax import lax
from jax.experimental import pallas as pl
from jax.experimental.pallas import tpu as pltpu
from jax.experimental.pallas import tpu_sc as plsc

_NC, _NS, _L = 2, 16, 16
_NW = _NC * _NS
_P = 128


@functools.partial(jax.jit, static_argnums=(3, 4, 5, 6, 7))
def _sc_interp(pair_table, posx, posy, B, C, H, W, N):
    NP = B * N
    n_chunks = NP // _P
    assert NP % _P == 0 and C % _L == 0
    per_w = -(-n_chunks // _NW)
    per_w += per_w % 2
    assert per_w * (_NW - 1) >= n_chunks - per_w
    HW = H * W
    fH, fW = float(H), float(W)
    sx = float(max(W - 1, 1))
    sy = float(max(H - 1, 1))
    mesh = plsc.VectorSubcoreMesh(core_axis_name="c", subcore_axis_name="s")

    @functools.partial(
        pl.kernel,
        out_type=jax.ShapeDtypeStruct((NP, C), jnp.float32),
        mesh=mesh,
        scratch_types=dict(
            px_v=pltpu.VMEM((2 * _P,), jnp.float32),
            py_v=pltpu.VMEM((2 * _P,), jnp.float32),
            idx_a=[pltpu.VMEM((_P,), jnp.int32) for _ in range(2)],
            idx_b=[pltpu.VMEM((_P,), jnp.int32) for _ in range(2)],
            w_a=[pltpu.VMEM((_P,), jnp.float32) for _ in range(4)],
            w_b=[pltpu.VMEM((_P,), jnp.float32) for _ in range(4)],
            g_a=[pltpu.VMEM((_P, 2 * C), jnp.float32) for _ in range(2)],
            g_b=[pltpu.VMEM((_P, 2 * C), jnp.float32) for _ in range(2)],
            o_a=pltpu.VMEM((_P, C), jnp.float32),
            o_b=pltpu.VMEM((_P, C), jnp.float32),
            gsem_a=pltpu.SemaphoreType.DMA,
            gsem_b=pltpu.SemaphoreType.DMA,
            osem_a=pltpu.SemaphoreType.DMA,
            osem_b=pltpu.SemaphoreType.DMA,
        ),
    )
    def kern(table_hbm, posx_hbm, posy_hbm, out_hbm,
             px_v, py_v, idx_a, idx_b, w_a, w_b, g_a, g_b, o_a, o_b,
             gsem_a, gsem_b, osem_a, osem_b):
        wid = lax.axis_index("s") * _NC + lax.axis_index("c")
        start = jnp.minimum(wid * per_w, n_chunks - per_w)
        lane = lax.iota(jnp.int32, 16)

        def fetch_pos_pair(cpair):
            off = cpair * _P
            pltpu.sync_copy(posx_hbm.at[pl.ds(off, 2 * _P)], px_v)
            pltpu.sync_copy(posy_hbm.at[pl.ds(off, 2 * _P)], py_v)

        def index_phase(c, half, idx_v, w_v):
            for j in range(_P // _L):
                s = pl.ds(j * _L, _L)
                sp = pl.ds(half * _P + j * _L, _L)
                px = px_v[sp]
                py = py_v[sp]
                gx = 2.0 * (px / sx) - 1.0
                gy = 2.0 * (py / sy) - 1.0
                ix = ((gx + 1.0) * fW - 1.0) / 2.0
                iy = ((gy + 1.0) * fH - 1.0) / 2.0
                tx = ix.astype(jnp.int32)
                tx = jnp.where(tx.astype(jnp.float32) > ix, tx - 1, tx)
                ty = iy.astype(jnp.int32)
                ty = jnp.where(ty.astype(jnp.float32) > iy, ty - 1, ty)
                wx1 = ix - tx.astype(jnp.float32)
                wx0 = 1.0 - wx1
                wy1 = iy - ty.astype(jnp.float32)
                wy0 = 1.0 - wy1
                x1 = tx + 1
                y1 = ty + 1
                ax = (jnp.where((tx >= 0) & (tx < W), wx0, 0.0)
                      + jnp.where(tx == -1, wx1, 0.0))
                bx = jnp.where((tx >= 0) & (x1 < W), wx1, 0.0)
                ay0 = jnp.where((ty >= 0) & (ty < H), wy0, 0.0)
                ay1 = jnp.where((y1 >= 0) & (y1 < H), wy1, 0.0)
                x0c = jnp.clip(tx, 0, W - 1)
                y0c = jnp.clip(ty, 0, H - 1)
                y1c = jnp.clip(y1, 0, H - 1)
                p_vec = c * _P + j * _L + lane
                rowbase = jnp.zeros((16,), jnp.int32)
                for k in range(1, B):
                    rowbase = rowbase + jnp.where(p_vec >= k * N, HW, 0)
                idx_v[0][s] = rowbase + y0c * W + x0c
                idx_v[1][s] = rowbase + y1c * W + x0c
                w_v[0][s] = ax * ay0
                w_v[1][s] = bx * ay0
                w_v[2][s] = ax * ay1
                w_v[3][s] = bx * ay1

        def fire_gathers(idx_v, g_v, sem):
            pltpu.async_copy(table_hbm.at[idx_v[0]], g_v[0], sem)
            pltpu.async_copy(table_hbm.at[idx_v[1]], g_v[1], sem)

        def wait_gathers(idx_v, g_v, sem):
            pltpu.make_async_copy(table_hbm.at[idx_v[0]], g_v[0], sem).wait()
            pltpu.make_async_copy(table_hbm.at[idx_v[1]], g_v[1], sem).wait()

        def combine(g_v, w_v, o_v):
            def grp_body(jv, _):
                gbase = jv * _L
                sg = pl.ds(gbase, _L)
                wv = [w_v[k][sg] for k in range(4)]
                for ii in range(_L):
                    p = gbase + ii
                    a00, a10, a01, a11 = wv[0][ii], wv[1][ii], wv[2][ii], wv[3][ii]
                    for q in range(C // _L):
                        s0 = pl.ds(q * _L, _L)
                        s1 = pl.ds(C + q * _L, _L)
                        o_v[p, s0] = (g_v[0][p, s0] * a00 + g_v[0][p, s1] * a10
                                      + g_v[1][p, s0] * a01 + g_v[1][p, s1] * a11)
                return 0

            lax.fori_loop(0, _P // _L, grp_body, 0)

        def fire_out(c, o_v, sem):
            pltpu.async_copy(o_v, out_hbm.at[pl.ds(c * _P, _P)], sem)

        def wait_out(c, o_v, sem):
            pltpu.make_async_copy(o_v, out_hbm.at[pl.ds(c * _P, _P)], sem).wait()

        fetch_pos_pair(start)
        index_phase(start, 0, idx_a, w_a)
        fire_gathers(idx_a, g_a, gsem_a)

        n_pairs = per_w // 2

        def pair_body(i, _):
            ca = start + 2 * i
            cb = ca + 1
            index_phase(cb, 1, idx_b, w_b)
            fire_gathers(idx_b, g_b, gsem_b)
            wait_gathers(idx_a, g_a, gsem_a)

            @pl.when(i > 0)
            def _():
                wait_out(ca, o_a, osem_a)

            combine(g_a, w_a, o_a)
            fire_out(ca, o_a, osem_a)

            @pl.when(i < n_pairs - 1)
            def _():
                fetch_pos_pair(ca + 2)
                index_phase(ca + 2, 0, idx_a, w_a)
                fire_gathers(idx_a, g_a, gsem_a)

            wait_gathers(idx_b, g_b, gsem_b)

            @pl.when(i > 0)
            def _():
                wait_out(cb, o_b, osem_b)

            combine(g_b, w_b, o_b)
            fire_out(cb, o_b, osem_b)
            return 0

        lax.fori_loop(0, n_pairs, pair_body, 0)
        last = start + per_w - 2
        wait_out(last, o_a, osem_a)
        wait_out(last + 1, o_b, osem_b)

    return kern(pair_table, posx, posy)


def kernel(x, pos, height, width):
    B, C, H, W = x.shape
    N = pos.shape[1]
    t = jnp.transpose(x, (0, 2, 3, 1)).reshape(B * H * W, C)
    pair_table = jnp.concatenate([t, jnp.roll(t, -1, axis=0)], axis=1)
    posx = pos[:, :, 0].reshape(-1)
    posy = pos[:, :, 1].reshape(-1)
    out = _sc_interp(pair_table, posx, posy, B, C, H, W, N)
    return out.reshape(B, N, C)
